# baseline (speedup 1.0000x reference)
import functools
import jax
import jax.numpy as jnp
from jax import lax
from jax.experimental import pallas as pl
from jax.experimental.pallas import tpu as pltpu
from jax.experimental.pallas import tpu_sc as plsc

N = 10000
E = 320000
D = 128
H = 128
C = 40
CP = 48

NC, NS = 2, 16
NW = NC * NS
CH = 128
NCHUNK = 80
E_PAD = NW * NCHUNK * CH
N_ACC = 10112
ROWS_PT = N_ACC // NS


@functools.lru_cache(maxsize=None)
def _seg_sum_kernel(F, nbuf, nch0):
  nch1 = 2 * NCHUNK - nch0
  assert (nch0 // 2) % nbuf == 0 and (nch1 // 2) % nbuf == 0
  nslab = nch0 // 2
  mesh = plsc.VectorSubcoreMesh(
      core_axis_name="c", subcore_axis_name="s", num_cores=NC, num_subcores=NS)

  @functools.partial(
      pl.kernel,
      out_type=jax.ShapeDtypeStruct((NC, N_ACC, F), jnp.float32),
      mesh=mesh,
      scratch_types=(
          [
              pltpu.VMEM((nslab, CH), jnp.int32),
              pltpu.VMEM((nslab, CH), jnp.int32),
          ]
          + [pltpu.VMEM((CH, F), jnp.float32) for _ in range(nbuf)]
          + [pltpu.VMEM_SHARED((N_ACC, F), jnp.float32)]
          + [pltpu.SemaphoreType.DMA for _ in range(nbuf)]
      ),
      compiler_params=pltpu.CompilerParams(use_tc_tiling_on_sc=False),
  )
  def seg_sum(h_hbm, src_hbm, dst_hbm, out_hbm, src_v, dst_v, *rest):
    bufs = rest[:nbuf]
    acc = rest[nbuf]
    sems = rest[nbuf + 1:]
    c = lax.axis_index("c")
    s = lax.axis_index("s")

    zv = jnp.zeros((16,), jnp.float32)

    with jax.named_scope("zero"):
      def zrow(i, carry):
        for k in range(F // 16):
          bufs[0][i, pl.ds(k * 16, 16)] = zv
        return carry

      lax.fori_loop(0, CH, zrow, 0)

      base = s * ROWS_PT
      for r in range(ROWS_PT // CH):
        pltpu.sync_copy(bufs[0], acc.at[pl.ds(base + r * CH, CH)])
      rem = ROWS_PT % CH
      if rem:
        pltpu.sync_copy(bufs[0].at[pl.ds(0, rem)],
                        acc.at[pl.ds(base + (ROWS_PT // CH) * CH, rem)])

      plsc.subcore_barrier()

    def wait_gather(k):
      pltpu.make_async_copy(h_hbm.at[pl.ds(0, CH)], bufs[k], sems[k]).wait()

    def run(cbase, nch):
      nh = nch // 2
      for half in range(2):
        chunk0 = cbase + half * nh
        pltpu.sync_copy(src_hbm.at[pl.ds(chunk0, nh)], src_v.at[pl.ds(0, nh)])
        pltpu.sync_copy(dst_hbm.at[pl.ds(chunk0, nh)], dst_v.at[pl.ds(0, nh)])

        for k in range(nbuf):
          pltpu.async_copy(h_hbm.at[src_v.at[k]], bufs[k], sems[k])

        def body(q, carry):
          j = q * nbuf
          for k in range(nbuf):
            wait_gather(k)
            pltpu.async_copy(h_hbm.at[src_v.at[j + k + nbuf]], bufs[k], sems[k])
          return carry

        lax.fori_loop(0, nh // nbuf - 1, body, 0)

        j = nh - nbuf
        for k in range(nbuf):
          wait_gather(k)

    with jax.named_scope("edges"):
      @pl.when(c == 0)
      def _():
        run(s * nch0, nch0)

      @pl.when(c == 1)
      def _():
        run(NS * nch0 + s * nch1, nch1)

      plsc.subcore_barrier()

    with jax.named_scope("copyout"):
      pltpu.sync_copy(acc.at[pl.ds(base, ROWS_PT)],
                      out_hbm.at[c, pl.ds(base, ROWS_PT)])

  return seg_sum


def _mlp_body(x_ref, a0_ref, a1_ref, w1_ref, b1_ref, w2_ref, m_ref):
  t = x_ref[...] + a0_ref[...] + a1_ref[...]
  h = jnp.dot(t, w1_ref[...], preferred_element_type=jnp.float32) + b1_ref[...]
  h = jnp.maximum(h, 0.0)
  m_ref[...] = jnp.dot(h, w2_ref[...], preferred_element_type=jnp.float32)


def _mlp(x, a0, a1, w1, b1, w2p):
  blk = 1000
  grid = (N // blk,)
  return pl.pallas_call(
      _mlp_body,
      grid=grid,
      in_specs=[
          pl.BlockSpec((blk, D), lambda i: (i, 0)),
          pl.BlockSpec((blk, D), lambda i: (i, 0)),
          pl.BlockSpec((blk, D), lambda i: (i, 0)),
          pl.BlockSpec((D, H), lambda i: (0, 0)),
          pl.BlockSpec((1, H), lambda i: (0, 0)),
          pl.BlockSpec((H, CP), lambda i: (0, 0)),
      ],
      out_specs=pl.BlockSpec((blk, CP), lambda i: (i, 0)),
      out_shape=jax.ShapeDtypeStruct((N, CP), jnp.float32),
  )(x, a0, a1, w1, b1, w2p)


def _final_body(m_ref, a0_ref, a1_ref, b2_ref, o_ref):
  o_ref[...] = m_ref[...] + a0_ref[...] + a1_ref[...] + b2_ref[...]


def _final(m, a0, a1, b2p):
  blk = 1000
  grid = (N // blk,)
  return pl.pallas_call(
      _final_body,
      grid=grid,
      in_specs=[
          pl.BlockSpec((blk, CP), lambda i: (i, 0)),
          pl.BlockSpec((blk, CP), lambda i: (i, 0)),
          pl.BlockSpec((blk, CP), lambda i: (i, 0)),
          pl.BlockSpec((1, CP), lambda i: (0, 0)),
      ],
      out_specs=pl.BlockSpec((blk, CP), lambda i: (i, 0)),
      out_shape=jax.ShapeDtypeStruct((N, CP), jnp.float32),
  )(m, a0, a1, b2p)


def kernel(x, edge_index, W1, b1, W2, b2):
  src = edge_index[0].astype(jnp.int32)
  dst = edge_index[1].astype(jnp.int32)
  pad = E_PAD - E
  dump = N + jnp.arange(pad, dtype=jnp.int32) % (N_ACC - N)
  src_p = jnp.concatenate([src, jnp.zeros((pad,), jnp.int32)]).reshape(NW * NCHUNK, CH)
  dst_p = jnp.concatenate([dst, dump]).reshape(NW * NCHUNK, CH)

  w2p = jnp.pad(W2, ((0, 0), (0, CP - C)))
  b1r = b1.reshape(1, H)
  b2r = jnp.pad(b2, (0, CP - C)).reshape(1, CP)

  agg_x = _seg_sum_kernel(D, 2, 128)(x, src_p, dst_p)
  m = _mlp(x, agg_x[0], agg_x[1], W1, b1r, w2p)
  agg_m = _seg_sum_kernel(CP, 4, 104)(m, src_p, dst_p)
  out = _final(m, agg_m[0], agg_m[1], b2r)
  return out[:, :C]


# speedup vs baseline: 1.7932x; 1.7932x over previous
import functools
import jax
import jax.numpy as jnp
from jax import lax
from jax.experimental import pallas as pl
from jax.experimental.pallas import tpu as pltpu
from jax.experimental.pallas import tpu_sc as plsc

N = 10000
E = 320000
D = 128
H = 128
C = 40
CP = 64

NC, NS = 2, 16
CH = 128
NCHT = 160
QC = 32
E_PAD = NS * NCHT * CH
N_ACC = 10112
ROWS_PT = N_ACC // NS
TROWS_PT = N // NS


@functools.lru_cache(maxsize=None)
def _seg_sum_kernel(FH, nbuf):
  assert QC % nbuf == 0 and NCHT % QC == 0
  mesh = plsc.VectorSubcoreMesh(
      core_axis_name="c", subcore_axis_name="s", num_cores=NC, num_subcores=NS)

  @functools.partial(
      pl.kernel,
      out_type=jax.ShapeDtypeStruct((NC, N_ACC, FH), jnp.float32),
      mesh=mesh,
      scratch_types=(
          [
              pltpu.VMEM((QC, CH), jnp.int32),
              pltpu.VMEM((QC, CH), jnp.int32),
          ]
          + [pltpu.VMEM((CH, FH), jnp.float32) for _ in range(nbuf)]
          + [
              pltpu.VMEM_SHARED((N, FH), jnp.float32),
              pltpu.VMEM_SHARED((N_ACC, FH), jnp.float32),
          ]
          + [pltpu.SemaphoreType.DMA for _ in range(nbuf)]
      ),
      compiler_params=pltpu.CompilerParams(use_tc_tiling_on_sc=False),
  )
  def seg_sum(h2_hbm, src_hbm, dst_hbm, out_hbm, src_v, dst_v, *rest):
    bufs = rest[:nbuf]
    table = rest[nbuf]
    acc = rest[nbuf + 1]
    sems = rest[nbuf + 2:]
    c = lax.axis_index("c")
    s = lax.axis_index("s")

    zv = jnp.zeros((16,), jnp.float32)

    with jax.named_scope("stage"):
      def zrow(i, carry):
        for k in range(FH // 16):
          bufs[0][i, pl.ds(k * 16, 16)] = zv
        return carry

      lax.fori_loop(0, CH, zrow, 0)

      base = s * ROWS_PT
      for r in range(ROWS_PT // CH):
        pltpu.sync_copy(bufs[0], acc.at[pl.ds(base + r * CH, CH)])
      rem = ROWS_PT % CH
      if rem:
        pltpu.sync_copy(bufs[0].at[pl.ds(0, rem)],
                        acc.at[pl.ds(base + (ROWS_PT // CH) * CH, rem)])

      tbase = s * TROWS_PT
      pltpu.sync_copy(h2_hbm.at[c, pl.ds(tbase, TROWS_PT)],
                      table.at[pl.ds(tbase, TROWS_PT)])

      plsc.subcore_barrier()

    def wait_gather(k):
      pltpu.make_async_copy(table.at[pl.ds(0, CH)], bufs[k], sems[k]).wait()

    with jax.named_scope("edges"):
      for g in range(NCHT // QC):
        chunk0 = s * NCHT + g * QC
        pltpu.sync_copy(src_hbm.at[pl.ds(chunk0, QC)], src_v)
        pltpu.sync_copy(dst_hbm.at[pl.ds(chunk0, QC)], dst_v)

        for k in range(nbuf):
          pltpu.async_copy(table.at[src_v.at[k]], bufs[k], sems[k])

        def body(q, carry):
          j = q * nbuf
          for k in range(nbuf):
            wait_gather(k)
            pltpu.sync_copy(bufs[k], acc.at[dst_v.at[j + k]], add=True)
            pltpu.async_copy(table.at[src_v.at[j + k + nbuf]], bufs[k], sems[k])
          return carry

        lax.fori_loop(0, QC // nbuf - 1, body, 0)

        j = QC - nbuf
        for k in range(nbuf):
          wait_gather(k)
          pltpu.sync_copy(bufs[k], acc.at[dst_v.at[j + k]], add=True)

      plsc.subcore_barrier()

    with jax.named_scope("copyout"):
      pltpu.sync_copy(acc.at[pl.ds(base, ROWS_PT)],
                      out_hbm.at[c, pl.ds(base, ROWS_PT)])

  return seg_sum


def _mlp_body(x_ref, a0_ref, a1_ref, w1_ref, b1_ref, w2_ref, m_ref):
  agg = jnp.concatenate([a0_ref[...], a1_ref[...]], axis=1)
  t = x_ref[...] + agg
  h = jnp.dot(t, w1_ref[...], preferred_element_type=jnp.float32) + b1_ref[...]
  h = jnp.maximum(h, 0.0)
  m_ref[...] = jnp.dot(h, w2_ref[...], preferred_element_type=jnp.float32)


def _mlp(x, a0, a1, w1, b1, w2p):
  blk = 1000
  grid = (N // blk,)
  hd = D // 2
  return pl.pallas_call(
      _mlp_body,
      grid=grid,
      in_specs=[
          pl.BlockSpec((blk, D), lambda i: (i, 0)),
          pl.BlockSpec((blk, hd), lambda i: (i, 0)),
          pl.BlockSpec((blk, hd), lambda i: (i, 0)),
          pl.BlockSpec((D, H), lambda i: (0, 0)),
          pl.BlockSpec((1, H), lambda i: (0, 0)),
          pl.BlockSpec((H, CP), lambda i: (0, 0)),
      ],
      out_specs=pl.BlockSpec((blk, CP), lambda i: (i, 0)),
      out_shape=jax.ShapeDtypeStruct((N, CP), jnp.float32),
  )(x, a0, a1, w1, b1, w2p)


def _final_body(m_ref, a0_ref, a1_ref, b2_ref, o_ref):
  agg = jnp.concatenate([a0_ref[...], a1_ref[...]], axis=1)
  o_ref[...] = m_ref[...] + agg + b2_ref[...]


def _final(m, a0, a1, b2p):
  blk = 1000
  grid = (N // blk,)
  hc = CP // 2
  return pl.pallas_call(
      _final_body,
      grid=grid,
      in_specs=[
          pl.BlockSpec((blk, CP), lambda i: (i, 0)),
          pl.BlockSpec((blk, hc), lambda i: (i, 0)),
          pl.BlockSpec((blk, hc), lambda i: (i, 0)),
          pl.BlockSpec((1, CP), lambda i: (0, 0)),
      ],
      out_specs=pl.BlockSpec((blk, CP), lambda i: (i, 0)),
      out_shape=jax.ShapeDtypeStruct((N, CP), jnp.float32),
  )(m, a0, a1, b2p)


def kernel(x, edge_index, W1, b1, W2, b2):
  src = edge_index[0].astype(jnp.int32)
  dst = edge_index[1].astype(jnp.int32)
  pad = E_PAD - E
  dump = N + jnp.arange(pad, dtype=jnp.int32) % (N_ACC - N)
  src_p = jnp.concatenate([src, jnp.zeros((pad,), jnp.int32)]).reshape(-1, CH)
  dst_p = jnp.concatenate([dst, dump]).reshape(-1, CH)

  w2p = jnp.pad(W2, ((0, 0), (0, CP - C)))
  b1r = b1.reshape(1, H)
  b2r = jnp.pad(b2, (0, CP - C)).reshape(1, CP)

  x2 = x.reshape(N, NC, D // 2).transpose(1, 0, 2)
  agg_x = _seg_sum_kernel(D // 2, 4)(x2, src_p, dst_p)
  m = _mlp(x, agg_x[0], agg_x[1], W1, b1r, w2p)
  m2 = m.reshape(N, NC, CP // 2).transpose(1, 0, 2)
  agg_m = _seg_sum_kernel(CP // 2, 8)(m2, src_p, dst_p)
  out = _final(m, agg_m[0], agg_m[1], b2r)
  return out[:, :C]


# speedup vs baseline: 1.9580x; 1.0919x over previous
import functools
import jax
import jax.numpy as jnp
from jax import lax
from jax.experimental import pallas as pl
from jax.experimental.pallas import tpu as pltpu
from jax.experimental.pallas import tpu_sc as plsc

N = 10000
E = 320000
D = 128
H = 128
C = 40
CP = 64

NC, NS = 2, 16
CH = 128
NCHT = 160
QC = 32
E_PAD = NS * NCHT * CH
N_ACC = 10112
ROWS_PT = N_ACC // NS
TROWS_PT = N // NS


@functools.lru_cache(maxsize=None)
def _seg_sum_kernel(FH, nbuf):
  assert QC % nbuf == 0 and NCHT % QC == 0
  mesh = plsc.VectorSubcoreMesh(
      core_axis_name="c", subcore_axis_name="s", num_cores=NC, num_subcores=NS)

  @functools.partial(
      pl.kernel,
      out_type=jax.ShapeDtypeStruct((NC, N_ACC, FH), jnp.float32),
      mesh=mesh,
      scratch_types=(
          [
              pltpu.VMEM((QC, CH), jnp.int32),
              pltpu.VMEM((QC, CH), jnp.int32),
          ]
          + [pltpu.VMEM((CH, FH), jnp.float32) for _ in range(nbuf)]
          + [
              pltpu.VMEM_SHARED((N, FH), jnp.float32),
              pltpu.VMEM_SHARED((N_ACC, FH), jnp.float32),
          ]
          + [pltpu.SemaphoreType.DMA for _ in range(nbuf)]
      ),
      compiler_params=pltpu.CompilerParams(use_tc_tiling_on_sc=False),
  )
  def seg_sum(h2_hbm, src_hbm, dst_hbm, out_hbm, src_v, dst_v, *rest):
    bufs = rest[:nbuf]
    table = rest[nbuf]
    acc = rest[nbuf + 1]
    sems = rest[nbuf + 2:]
    c = lax.axis_index("c")
    s = lax.axis_index("s")

    zv = jnp.zeros((16,), jnp.float32)

    with jax.named_scope("stage"):
      def zrow(i, carry):
        for k in range(FH // 16):
          bufs[0][i, pl.ds(k * 16, 16)] = zv
        return carry

      lax.fori_loop(0, CH, zrow, 0)

      base = s * ROWS_PT
      for r in range(ROWS_PT // CH):
        pltpu.sync_copy(bufs[0], acc.at[pl.ds(base + r * CH, CH)])
      rem = ROWS_PT % CH
      if rem:
        pltpu.sync_copy(bufs[0].at[pl.ds(0, rem)],
                        acc.at[pl.ds(base + (ROWS_PT // CH) * CH, rem)])

      tbase = s * TROWS_PT
      pltpu.sync_copy(h2_hbm.at[pl.ds(tbase, TROWS_PT), pl.ds(c * FH, FH)],
                      table.at[pl.ds(tbase, TROWS_PT)])

      plsc.subcore_barrier()

    def wait_gather(k):
      pltpu.make_async_copy(table.at[pl.ds(0, CH)], bufs[k], sems[k]).wait()

    with jax.named_scope("edges"):
      for g in range(NCHT // QC):
        chunk0 = s * NCHT + g * QC
        pltpu.sync_copy(src_hbm.at[pl.ds(chunk0, QC)], src_v)
        pltpu.sync_copy(dst_hbm.at[pl.ds(chunk0, QC)], dst_v)

        for k in range(nbuf):
          pltpu.async_copy(table.at[src_v.at[k]], bufs[k], sems[k])

        def body(q, carry):
          j = q * nbuf
          for k in range(nbuf):
            wait_gather(k)
            pltpu.sync_copy(bufs[k], acc.at[dst_v.at[j + k]], add=True)
            pltpu.async_copy(table.at[src_v.at[j + k + nbuf]], bufs[k], sems[k])
          return carry

        lax.fori_loop(0, QC // nbuf - 1, body, 0)

        j = QC - nbuf
        for k in range(nbuf):
          wait_gather(k)
          pltpu.sync_copy(bufs[k], acc.at[dst_v.at[j + k]], add=True)

      plsc.subcore_barrier()

    with jax.named_scope("copyout"):
      pltpu.sync_copy(acc.at[pl.ds(base, ROWS_PT)],
                      out_hbm.at[c, pl.ds(base, ROWS_PT)])

  return seg_sum


def _mlp_body(x_ref, a0_ref, a1_ref, w1_ref, b1_ref, w2_ref, m_ref):
  agg = jnp.concatenate([a0_ref[...], a1_ref[...]], axis=1)
  t = x_ref[...] + agg
  h = jnp.dot(t, w1_ref[...], preferred_element_type=jnp.float32) + b1_ref[...]
  h = jnp.maximum(h, 0.0)
  m_ref[...] = jnp.dot(h, w2_ref[...], preferred_element_type=jnp.float32)


def _mlp(x, a0, a1, w1, b1, w2p):
  blk = 1000
  grid = (N // blk,)
  hd = D // 2
  return pl.pallas_call(
      _mlp_body,
      grid=grid,
      in_specs=[
          pl.BlockSpec((blk, D), lambda i: (i, 0)),
          pl.BlockSpec((blk, hd), lambda i: (i, 0)),
          pl.BlockSpec((blk, hd), lambda i: (i, 0)),
          pl.BlockSpec((D, H), lambda i: (0, 0)),
          pl.BlockSpec((1, H), lambda i: (0, 0)),
          pl.BlockSpec((H, CP), lambda i: (0, 0)),
      ],
      out_specs=pl.BlockSpec((blk, CP), lambda i: (i, 0)),
      out_shape=jax.ShapeDtypeStruct((N, CP), jnp.float32),
  )(x, a0, a1, w1, b1, w2p)


def _final_body(m_ref, a0_ref, a1_ref, b2_ref, o_ref):
  agg = jnp.concatenate([a0_ref[...], a1_ref[...]], axis=1)
  o_ref[...] = m_ref[...] + agg + b2_ref[...]


def _final(m, a0, a1, b2p):
  blk = 1000
  grid = (N // blk,)
  hc = CP // 2
  return pl.pallas_call(
      _final_body,
      grid=grid,
      in_specs=[
          pl.BlockSpec((blk, CP), lambda i: (i, 0)),
          pl.BlockSpec((blk, hc), lambda i: (i, 0)),
          pl.BlockSpec((blk, hc), lambda i: (i, 0)),
          pl.BlockSpec((1, CP), lambda i: (0, 0)),
      ],
      out_specs=pl.BlockSpec((blk, CP), lambda i: (i, 0)),
      out_shape=jax.ShapeDtypeStruct((N, CP), jnp.float32),
  )(m, a0, a1, b2p)


def kernel(x, edge_index, W1, b1, W2, b2):
  src = edge_index[0].astype(jnp.int32)
  dst = edge_index[1].astype(jnp.int32)
  pad = E_PAD - E
  dump = N + jnp.arange(pad, dtype=jnp.int32) % (N_ACC - N)
  src_p = jnp.concatenate([src, jnp.zeros((pad,), jnp.int32)]).reshape(-1, CH)
  dst_p = jnp.concatenate([dst, dump]).reshape(-1, CH)

  w2p = jnp.pad(W2, ((0, 0), (0, CP - C)))
  b1r = b1.reshape(1, H)
  b2r = jnp.pad(b2, (0, CP - C)).reshape(1, CP)

  agg_x = _seg_sum_kernel(D // 2, 4)(x, src_p, dst_p)
  m = _mlp(x, agg_x[0], agg_x[1], W1, b1r, w2p)
  agg_m = _seg_sum_kernel(CP // 2, 8)(m, src_p, dst_p)
  out = _final(m, agg_m[0], agg_m[1], b2r)
  return out[:, :C]
